# baseline (speedup 1.0000x reference)
import jax
import jax.numpy as jnp
import numpy as np
from jax import lax
from jax.experimental import pallas as pl
from jax.experimental.pallas import tpu as pltpu
from jax.experimental.pallas import tpu_sc as plsc

_VOCAB = 1000000
_EMBED = 64
_MAXLEN = 200
_BATCH = 4096

_L = 16
_NW = 32
_SEQ_PER_W = _BATCH // _NW
_BG = _BATCH // _SEQ_PER_W

_TB = 128
_NFULL = _VOCAB // _TB
_TAIL = _VOCAB - _NFULL * _TB
_BLK_PER_W = (_NFULL + _NW - 1) // _NW


def _repack_body(tokT_hbm, flat_hbm, buf_v, out_v, tail_v, tailo_v):
    wid = lax.axis_index("s") * 2 + lax.axis_index("c")

    @pl.loop(0, _BLK_PER_W)
    def _blk(i):
        j = wid + i * _NW

        @pl.when(j < _NFULL)
        def _():
            pltpu.sync_copy(tokT_hbm.at[:, pl.ds(j * _TB, _TB)], buf_v)
            for e in range(_EMBED):
                for k in range(_TB // _L):
                    v = buf_v[e, pl.ds(k * _L, _L)]
                    plsc.store_scatter(
                        out_v,
                        [(k * _L + lax.iota(jnp.int32, _L)) * _EMBED + e],
                        v,
                    )
            pltpu.sync_copy(
                out_v, flat_hbm.at[pl.ds(j * (_TB * _EMBED), _TB * _EMBED)]
            )

    @pl.when(wid == 0)
    def _tail():
        pltpu.sync_copy(tokT_hbm.at[:, pl.ds(_NFULL * _TB, _TAIL)], tail_v)
        for e in range(_EMBED):
            for k in range(_TAIL // _L):
                v = tail_v[e, pl.ds(k * _L, _L)]
                plsc.store_scatter(
                    tailo_v,
                    [(k * _L + lax.iota(jnp.int32, _L)) * _EMBED + e],
                    v,
                )
        pltpu.sync_copy(
            tailo_v,
            flat_hbm.at[pl.ds(_NFULL * _TB * _EMBED, _TAIL * _EMBED)],
        )


@jax.jit
def _repack_call(tokT):
    return pl.kernel(
        _repack_body,
        out_type=jax.ShapeDtypeStruct((_VOCAB * _EMBED,), jnp.float32),
        mesh=plsc.VectorSubcoreMesh(core_axis_name="c", subcore_axis_name="s"),
        compiler_params=pltpu.CompilerParams(
            use_tc_tiling_on_sc=True, needs_layout_passes=False
        ),
        scratch_types=[
            pltpu.VMEM((_EMBED, _TB), jnp.float32),
            pltpu.VMEM((_TB * _EMBED,), jnp.float32),
            pltpu.VMEM((_EMBED, _TAIL), jnp.float32),
            pltpu.VMEM((_TAIL * _EMBED,), jnp.float32),
        ],
    )(tokT)


_IDS_PER_W = _SEQ_PER_W * _MAXLEN
_KB = _SEQ_PER_W // _L


def _gather_body(
    seq_hbm, tok_hbm, pos_hbm, aux_hbm, out_hbm,
    idx_v, pos_v, aux_v, idxrow_v, rows_v, buf_v, sem,
):
    wid = lax.axis_index("s") * 2 + lax.axis_index("c")

    pltpu.sync_copy(pos_hbm, pos_v)
    pltpu.sync_copy(aux_hbm, aux_v)
    pltpu.sync_copy(seq_hbm.at[pl.ds(wid * _IDS_PER_W, _IDS_PER_W)], idx_v)

    @pl.loop(0, _MAXLEN)
    def _pos(l):
        base = aux_v[l, 0, :]
        for k in range(_KB):
            idxrow_v[pl.ds(k * _L, _L)] = plsc.load_gather(
                idx_v, [base + k * (_L * _MAXLEN)]
            )

        pltpu.async_copy(tok_hbm.at[idxrow_v], rows_v, sem).wait()

        for k in range(_KB):
            ids = idxrow_v[pl.ds(k * _L, _L)]
            mask = ids == 0
            nzero = jnp.sum(jnp.where(mask, 1, 0))

            @pl.when(nzero > 0)
            def _():
                rows = k * _L + lax.iota(jnp.int32, _L)
                zeros = jnp.zeros((_L,), jnp.float32)
                for e in range(_EMBED):
                    plsc.store_scatter(
                        rows_v,
                        [rows, jnp.full((_L,), e, jnp.int32)],
                        zeros,
                        mask=mask,
                    )

        for e in range(_EMBED):
            pvec = plsc.load_gather(pos_v, [aux_v[l, 1, :] + e])
            for k in range(_KB):
                tv = plsc.load_gather(
                    rows_v,
                    [
                        k * _L + lax.iota(jnp.int32, _L),
                        jnp.full((_L,), e, jnp.int32),
                    ],
                )
                buf_v[e // 8, e % 8, pl.ds(k * _L, _L)] = tv + pvec

        for eg in range(_EMBED // 8):
            pltpu.sync_copy(buf_v.at[eg], out_hbm.at[l, eg, wid])


@jax.jit
def _gather_call(seq_flat, tok_lin, pos_flat, aux):
    return pl.kernel(
        _gather_body,
        out_type=jax.ShapeDtypeStruct(
            (_MAXLEN, _EMBED // 8, _BG, 8, _SEQ_PER_W), jnp.float32
        ),
        mesh=plsc.VectorSubcoreMesh(core_axis_name="c", subcore_axis_name="s"),
        compiler_params=pltpu.CompilerParams(
            use_tc_tiling_on_sc=False, needs_layout_passes=False
        ),
        scratch_types=[
            pltpu.VMEM((_IDS_PER_W,), jnp.int32),
            pltpu.VMEM((_MAXLEN * _EMBED,), jnp.float32),
            pltpu.VMEM((_MAXLEN, 2, _L), jnp.int32),
            pltpu.VMEM((_SEQ_PER_W,), jnp.int32),
            pltpu.VMEM((_SEQ_PER_W, _EMBED), jnp.float32),
            pltpu.VMEM((_EMBED // 8, 8, _SEQ_PER_W), jnp.float32),
            pltpu.SemaphoreType.DMA,
        ],
    )(seq_flat, tok_lin, pos_flat, aux)


_AUX = np.stack(
    [
        np.arange(_L, dtype=np.int32)[None, :] * _MAXLEN
        + np.arange(_MAXLEN, dtype=np.int32)[:, None],
        np.broadcast_to(
            (np.arange(_MAXLEN, dtype=np.int32) * _EMBED)[:, None], (_MAXLEN, _L)
        ),
    ],
    axis=1,
)


def kernel(sequence, token_table, pos_table):
    tok_flat = _repack_call(token_table.T)
    seq_flat = sequence.reshape(_BATCH * _MAXLEN)
    pos_flat = pos_table.reshape(_MAXLEN * _EMBED)
    tok_lin = tok_flat.reshape(_VOCAB, _EMBED)
    out5 = _gather_call(seq_flat, tok_lin, pos_flat, jnp.asarray(_AUX))
    return out5.transpose(2, 4, 0, 1, 3).reshape(_BATCH, _MAXLEN, _EMBED)


# speedup vs baseline: 2.0723x; 2.0723x over previous
import jax
import jax.numpy as jnp
import numpy as np
from jax import lax
from jax.experimental import pallas as pl
from jax.experimental.pallas import tpu as pltpu
from jax.experimental.pallas import tpu_sc as plsc

_VOCAB = 1000000
_EMBED = 64
_PITCH = 72
_MAXLEN = 200
_BATCH = 4096

_L = 16
_NW = 32
_SEQ_PER_W = _BATCH // _NW
_BG = _BATCH // _SEQ_PER_W

_TB = 256
_NBLK = _VOCAB // _TB
_TAIL = _VOCAB - _NBLK * _TB
_PAIRS = (_NBLK // _NW + 2) // 2


def _repack_body(
    tokT_hbm, flat_hbm, buf_a, buf_b, out_a, out_b, tail_v, tailo_v, semi, semo
):
    wid = lax.axis_index("s") * 2 + lax.axis_index("c")
    bufs = (buf_a, buf_b)
    outs = (out_a, out_b)

    def fire_in(i, b):
        j = wid + i * _NW

        @pl.when(j < _NBLK)
        def _():
            pltpu.async_copy(
                tokT_hbm.at[:, pl.ds(j * _TB, _TB)], bufs[b], semi[b]
            )

    fire_in(0, 0)

    @pl.loop(0, _PAIRS)
    def _pair(it):
        for p in range(2):
            i = it * 2 + p
            j = wid + i * _NW

            @pl.when(j < _NBLK)
            def _():
                fire_in(i + 1, 1 - p)
                pltpu.make_async_copy(
                    tokT_hbm.at[:, pl.ds(0, _TB)], bufs[p], semi[p]
                ).wait()

                @pl.when(i >= 2)
                def _():
                    pltpu.make_async_copy(
                        flat_hbm.at[pl.ds(0, _TB * _PITCH)], outs[p], semo[p]
                    ).wait()

                for e in range(_EMBED):
                    for k in range(_TB // _L):
                        v = bufs[p][e, pl.ds(k * _L, _L)]
                        plsc.store_scatter(
                            outs[p],
                            [(k * _L + lax.iota(jnp.int32, _L)) * _PITCH + e],
                            v,
                        )
                pltpu.async_copy(
                    outs[p],
                    flat_hbm.at[pl.ds(j * (_TB * _PITCH), _TB * _PITCH)],
                    semo[p],
                )

    for p in range(2):
        pltpu.make_async_copy(
            flat_hbm.at[pl.ds(0, _TB * _PITCH)], outs[p], semo[p]
        ).wait()

    @pl.when(wid == 0)
    def _tail():
        pltpu.sync_copy(tokT_hbm.at[:, pl.ds(_NBLK * _TB, _TAIL)], tail_v)
        for e in range(_EMBED):
            for k in range(_TAIL // _L):
                v = tail_v[e, pl.ds(k * _L, _L)]
                plsc.store_scatter(
                    tailo_v,
                    [(k * _L + lax.iota(jnp.int32, _L)) * _PITCH + e],
                    v,
                )
        pltpu.sync_copy(
            tailo_v,
            flat_hbm.at[pl.ds(_NBLK * _TB * _PITCH, _TAIL * _PITCH)],
        )


@jax.jit
def _repack_call(tokT):
    return pl.kernel(
        _repack_body,
        out_type=jax.ShapeDtypeStruct((_VOCAB * _PITCH,), jnp.float32),
        mesh=plsc.VectorSubcoreMesh(core_axis_name="c", subcore_axis_name="s"),
        compiler_params=pltpu.CompilerParams(
            use_tc_tiling_on_sc=True, needs_layout_passes=False
        ),
        scratch_types=[
            pltpu.VMEM((_EMBED, _TB), jnp.float32),
            pltpu.VMEM((_EMBED, _TB), jnp.float32),
            pltpu.VMEM((_TB * _PITCH,), jnp.float32),
            pltpu.VMEM((_TB * _PITCH,), jnp.float32),
            pltpu.VMEM((_EMBED, _TAIL), jnp.float32),
            pltpu.VMEM((_TAIL * _PITCH,), jnp.float32),
            [pltpu.SemaphoreType.DMA, pltpu.SemaphoreType.DMA],
            [pltpu.SemaphoreType.DMA, pltpu.SemaphoreType.DMA],
        ],
    )(tokT)


_IDS_PER_W = _SEQ_PER_W * _MAXLEN
_KB = _SEQ_PER_W // _L


def _gather_body(
    seq_hbm, tok_hbm, pos_hbm, aux_hbm, out_hbm,
    idx_v, pos_v, aux_v, idxrows, rows, bufs, semg, semo,
):
    wid = lax.axis_index("s") * 2 + lax.axis_index("c")

    pltpu.sync_copy(pos_hbm, pos_v)
    pltpu.sync_copy(aux_hbm, aux_v)
    pltpu.sync_copy(seq_hbm.at[pl.ds(wid * _IDS_PER_W, _IDS_PER_W)], idx_v)

    def assemble_and_fire(l, b):
        base = aux_v[l, 0, :]
        for k in range(_KB):
            idxrows[b, pl.ds(k * _L, _L)] = plsc.load_gather(
                idx_v, [base + k * (_L * _MAXLEN)]
            )
        pltpu.async_copy(tok_hbm.at[idxrows.at[b]], rows.at[b], semg[b])

    assemble_and_fire(0, 0)

    @pl.loop(0, _MAXLEN // 2)
    def _pair(it):
        for p in range(2):
            l = it * 2 + p
            lnext = jnp.minimum(l + 1, _MAXLEN - 1)
            assemble_and_fire(lnext, 1 - p)
            pltpu.make_async_copy(
                tok_hbm.at[pl.ds(0, _SEQ_PER_W)], rows.at[p], semg[p]
            ).wait()

            for k in range(_KB):
                ids = idxrows[p, pl.ds(k * _L, _L)]
                mask = ids == 0
                nzero = jnp.sum(jnp.where(mask, 1, 0))

                @pl.when(nzero > 0)
                def _():
                    rws = k * _L + lax.iota(jnp.int32, _L)
                    zeros = jnp.zeros((_L,), jnp.float32)
                    for e in range(_EMBED):
                        plsc.store_scatter(
                            rows.at[p],
                            [rws, jnp.full((_L,), e, jnp.int32)],
                            zeros,
                            mask=mask,
                        )

            @pl.when(l >= 2)
            def _():
                for eg in range(_EMBED // 8):
                    pltpu.make_async_copy(
                        out_hbm.at[0, 0, 0], bufs.at[p, eg], semo[p]
                    ).wait()

            for e in range(_EMBED):
                pvec = plsc.load_gather(pos_v, [aux_v[l, 1, :] + e])
                for k in range(_KB):
                    tv = plsc.load_gather(
                        rows.at[p],
                        [
                            k * _L + lax.iota(jnp.int32, _L),
                            jnp.full((_L,), e, jnp.int32),
                        ],
                    )
                    bufs[p, e // 8, e % 8, pl.ds(k * _L, _L)] = tv + pvec

            for eg in range(_EMBED // 8):
                pltpu.async_copy(
                    bufs.at[p, eg], out_hbm.at[l, eg, wid], semo[p]
                )

    for p in range(2):
        for eg in range(_EMBED // 8):
            pltpu.make_async_copy(
                out_hbm.at[0, 0, 0], bufs.at[p, eg], semo[p]
            ).wait()
    pltpu.make_async_copy(
        tok_hbm.at[pl.ds(0, _SEQ_PER_W)], rows.at[0], semg[0]
    ).wait()


@jax.jit
def _gather_call(seq_flat, tok_lin, pos_flat, aux):
    return pl.kernel(
        _gather_body,
        out_type=jax.ShapeDtypeStruct(
            (_MAXLEN, _EMBED // 8, _BG, 8, _SEQ_PER_W), jnp.float32
        ),
        mesh=plsc.VectorSubcoreMesh(core_axis_name="c", subcore_axis_name="s"),
        compiler_params=pltpu.CompilerParams(
            use_tc_tiling_on_sc=False, needs_layout_passes=False
        ),
        scratch_types=[
            pltpu.VMEM((_IDS_PER_W,), jnp.int32),
            pltpu.VMEM((_MAXLEN * _EMBED,), jnp.float32),
            pltpu.VMEM((_MAXLEN, 2, _L), jnp.int32),
            pltpu.VMEM((2, _SEQ_PER_W), jnp.int32),
            pltpu.VMEM((2, _SEQ_PER_W, _PITCH), jnp.float32),
            pltpu.VMEM((2, _EMBED // 8, 8, _SEQ_PER_W), jnp.float32),
            [pltpu.SemaphoreType.DMA, pltpu.SemaphoreType.DMA],
            [pltpu.SemaphoreType.DMA, pltpu.SemaphoreType.DMA],
        ],
    )(seq_flat, tok_lin, pos_flat, aux)


_AUX = np.stack(
    [
        np.arange(_L, dtype=np.int32)[None, :] * _MAXLEN
        + np.arange(_MAXLEN, dtype=np.int32)[:, None],
        np.broadcast_to(
            (np.arange(_MAXLEN, dtype=np.int32) * _EMBED)[:, None], (_MAXLEN, _L)
        ),
    ],
    axis=1,
)


def kernel(sequence, token_table, pos_table):
    tok_flat = _repack_call(token_table.T)
    seq_flat = sequence.reshape(_BATCH * _MAXLEN)
    pos_flat = pos_table.reshape(_MAXLEN * _EMBED)
    tok_lin = tok_flat.reshape(_VOCAB, _PITCH)
    out5 = _gather_call(seq_flat, tok_lin, pos_flat, jnp.asarray(_AUX))
    return out5.transpose(2, 4, 0, 1, 3).reshape(_BATCH, _MAXLEN, _EMBED)


# speedup vs baseline: 2.3870x; 1.1518x over previous
import jax
import jax.numpy as jnp
from jax import lax
from jax.experimental import pallas as pl
from jax.experimental.pallas import tpu as pltpu
from jax.experimental.pallas import tpu_sc as plsc

_VOCAB = 1000000
_EMBED = 64
_MAXLEN = 200
_BATCH = 4096

_L = 16
_NW = 32
_EC = _EMBED // _L
_SEQ_PER_W = _BATCH // _NW
_CS = 2
_ROWS = _CS * _MAXLEN
_NCH = _SEQ_PER_W // _CS
_GSPLITS = ((0, 128), (128, 128), (256, 128), (384, 16))
_IDX_VREGS = _ROWS // _L


def _emb_body(seq_hbm, tok_hbm, pos_hbm, out_hbm, idx_v, rows_v, pos_v, sem):
    wid = lax.axis_index("s") * 2 + lax.axis_index("c")
    row0 = wid * (_SEQ_PER_W * _MAXLEN)

    pltpu.sync_copy(pos_hbm, pos_v)

    @pl.loop(0, _NCH)
    def _chunk(g):
        base = row0 + g * _ROWS
        pltpu.sync_copy(seq_hbm.at[pl.ds(base, _ROWS)], idx_v)

        cps = [
            pltpu.async_copy(
                tok_hbm.at[idx_v.at[pl.ds(off, num)]],
                rows_v.at[pl.ds(off, num)],
                sem,
            )
            for off, num in _GSPLITS
        ]
        for cp in cps:
            cp.wait()

        @pl.loop(0, _IDX_VREGS)
        def _fix(m):
            v = idx_v[pl.ds(m * _L, _L)]
            mask = v == 0
            nzero = jnp.sum(jnp.where(mask, 1, 0))

            @pl.when(nzero > 0)
            def _():
                rows = lax.iota(jnp.int32, _L)
                zeros = jnp.zeros((_L,), jnp.float32)
                tile = rows_v.at[pl.ds(m * _L, _L), :]
                for col in range(_EMBED):
                    plsc.store_scatter(
                        tile,
                        [rows, jnp.full((_L,), col, jnp.int32)],
                        zeros,
                        mask=mask,
                    )

        @pl.loop(0, _MAXLEN)
        def _add(l):
            for s in range(_CS):
                r = s * _MAXLEN + l
                for c in range(_EC):
                    plsc.addupdate(
                        rows_v.at[r, pl.ds(c * _L, _L)],
                        pos_v[l, pl.ds(c * _L, _L)],
                    )

        pltpu.sync_copy(rows_v, out_hbm.at[pl.ds(base, _ROWS)])


@jax.jit
def _emb_call(seq_flat, tok_table, pos_table):
    return pl.kernel(
        _emb_body,
        out_type=jax.ShapeDtypeStruct((_BATCH * _MAXLEN, _EMBED), jnp.float32),
        mesh=plsc.VectorSubcoreMesh(core_axis_name="c", subcore_axis_name="s"),
        compiler_params=pltpu.CompilerParams(
            use_tc_tiling_on_sc=False, needs_layout_passes=False
        ),
        scratch_types=[
            pltpu.VMEM((_ROWS,), jnp.int32),
            pltpu.VMEM((_ROWS, _EMBED), jnp.float32),
            pltpu.VMEM((_MAXLEN, _EMBED), jnp.float32),
            pltpu.SemaphoreType.DMA,
        ],
    )(seq_flat, tok_table, pos_table)


def kernel(sequence, token_table, pos_table):
    seq_flat = sequence.reshape(_BATCH * _MAXLEN)
    out = _emb_call(seq_flat, token_table, pos_table)
    return out.reshape(_BATCH, _MAXLEN, _EMBED)
